# baseline (speedup 1.0000x reference)
import functools

import jax
import jax.numpy as jnp
import numpy as np
from jax.experimental import pallas as pl

_TOPK_RATIO = 0.1
_MININT = np.int32(-(2 ** 31))


def _fused_mm_kernel(x_ref, w1_ref, b1_ref, wa1_ref, ba1_ref, w2_ref, b2_ref,
                     wa2_ref, ba2_ref, seg_ref, attn_ref):
    x = x_ref[...]
    t1 = jax.lax.dot_general(x, w1_ref[...], (((1,), (1,)), ((), ())),
                             preferred_element_type=jnp.float32)
    h = jnp.maximum(t1 + b1_ref[...], 0.0)
    seg = jax.lax.dot_general(h, w2_ref[...], (((1,), (1,)), ((), ())),
                              preferred_element_type=jnp.float32) + b2_ref[...]
    seg_ref[...] = seg
    t2 = jax.lax.dot_general(x, wa1_ref[...], (((1,), (1,)), ((), ())),
                             preferred_element_type=jnp.float32)
    ha = jnp.tanh(t2 + ba1_ref[...])
    a = jax.lax.dot_general(ha, wa2_ref[...], (((1,), (1,)), ((), ())),
                            preferred_element_type=jnp.float32)
    attn_ref[...] = a[:, 0:1] + ba2_ref[0, 0]


def _weights_kernel(attn_ref, w_ref, *, k):
    a = attn_ref[...]
    B, T = a.shape
    bits = jax.lax.bitcast_convert_type(a, jnp.int32)
    sk = jnp.where(bits < 0,
                   jnp.bitwise_xor(jnp.bitwise_not(bits), _MININT),
                   bits)

    def body(i, p):
        b = jnp.int32(31) - i
        cand = jnp.bitwise_or(p, jnp.left_shift(jnp.int32(1), b))
        icand = jnp.bitwise_xor(cand, _MININT)
        cnt = jnp.sum((sk >= icand).astype(jnp.int32), axis=1, keepdims=True)
        return jnp.where(cnt >= k, cand, p)

    p = jax.lax.fori_loop(0, 32, body, jnp.zeros((B, 1), jnp.int32))
    ithr = jnp.bitwise_xor(p, _MININT)

    gt = sk > ithr
    c_gt = jnp.sum(gt.astype(jnp.int32), axis=1, keepdims=True)
    eq = sk == ithr
    r = jnp.int32(k) - c_gt
    e = eq.astype(jnp.int32)
    s = 1
    while s < T:
        e = e + jnp.concatenate(
            [jnp.zeros((B, s), jnp.int32), e[:, :T - s]], axis=1)
        s *= 2
    sel = jnp.logical_or(gt, jnp.logical_and(eq, e <= r))
    mask = jnp.where(sel, jnp.float32(1.0 / k), jnp.float32(0.0))
    ssum = jnp.sum(mask, axis=1, keepdims=True)
    w_ref[...] = mask / (ssum + jnp.float32(1e-8))


def _pool_kernel(w_ref, seg_ref, clip_ref):
    w = w_ref[0]
    seg = seg_ref[0]
    clip_ref[0] = jax.lax.dot_general(w, seg, (((1,), (0,)), ((), ())),
                                      preferred_element_type=jnp.float32)


def kernel(x, W1, b1, W2, b2, Wa1, ba1, Wa2, ba2):
    B, T, D = x.shape
    HID = W1.shape[0]
    C = W2.shape[0]
    k = max(1, min(T, int(round(T * _TOPK_RATIO))))
    M = B * T
    TM = 512 if M % 512 == 0 else T

    xf = x.reshape(M, D)
    b1r = b1.reshape(1, HID)
    ba1r = ba1.reshape(1, HID)
    b2r = b2.reshape(1, C)
    ba2r = ba2.reshape(1, 1)
    wa2p = jnp.zeros((128, HID), jnp.float32).at[0].set(Wa2[0])

    seg_flat, attn_flat = pl.pallas_call(
        _fused_mm_kernel,
        grid=(M // TM,),
        in_specs=[
            pl.BlockSpec((TM, D), lambda i: (i, 0)),
            pl.BlockSpec((HID, D), lambda i: (0, 0)),
            pl.BlockSpec((1, HID), lambda i: (0, 0)),
            pl.BlockSpec((HID, D), lambda i: (0, 0)),
            pl.BlockSpec((1, HID), lambda i: (0, 0)),
            pl.BlockSpec((C, HID), lambda i: (0, 0)),
            pl.BlockSpec((1, C), lambda i: (0, 0)),
            pl.BlockSpec((128, HID), lambda i: (0, 0)),
            pl.BlockSpec((1, 1), lambda i: (0, 0)),
        ],
        out_specs=[
            pl.BlockSpec((TM, C), lambda i: (i, 0)),
            pl.BlockSpec((TM, 1), lambda i: (i, 0)),
        ],
        out_shape=[
            jax.ShapeDtypeStruct((M, C), jnp.float32),
            jax.ShapeDtypeStruct((M, 1), jnp.float32),
        ],
    )(xf, W1, b1r, Wa1, ba1r, W2, b2r, wa2p, ba2r)

    seg_logits = seg_flat.reshape(B, T, C)
    attn = attn_flat.reshape(B, T)

    weights = pl.pallas_call(
        functools.partial(_weights_kernel, k=k),
        grid=(1,),
        in_specs=[pl.BlockSpec((B, T), lambda i: (0, 0))],
        out_specs=[pl.BlockSpec((B, T), lambda i: (0, 0))],
        out_shape=[jax.ShapeDtypeStruct((B, T), jnp.float32)],
    )(attn)[0]

    clip_logits = pl.pallas_call(
        _pool_kernel,
        grid=(B,),
        in_specs=[
            pl.BlockSpec((1, 1, T), lambda b: (b, 0, 0)),
            pl.BlockSpec((1, T, C), lambda b: (b, 0, 0)),
        ],
        out_specs=[pl.BlockSpec((1, 1, C), lambda b: (b, 0, 0))],
        out_shape=[jax.ShapeDtypeStruct((B, 1, C), jnp.float32)],
    )(weights.reshape(B, 1, T), seg_logits)[0]

    return clip_logits.reshape(B, C), seg_logits, weights
